# Initial kernel scaffold; baseline (speedup 1.0000x reference)
import jax
import jax.numpy as jnp
from jax.experimental import pallas as pl


def kernel(x, edge_index, W0, b0, g0, be0, W1, b1, g1, be1, W2, b2):
    raise NotImplementedError("write your pallas kernel here")


# speedup vs baseline: 6.5698x; 6.5698x over previous
import functools

import jax
import jax.numpy as jnp
from jax import lax
from jax.experimental import pallas as pl
from jax.experimental.pallas import tpu as pltpu
from jax.experimental.pallas import tpu_sc as plsc

N = 10000
E = 320000
D = 128
NCLS = 40
NPAD = 10240
NC, NS = 2, 16
KC = 200
ROWS_PER_TILE = NPAD // NS

_MESH = dict(core_axis_name="c", subcore_axis_name="s")


def _deg_body(ei_hbm, ones_hbm, zeros_hbm, out_hbm, acc_sh, idx_v, ones_v, sem):
    c = lax.axis_index("c")
    s = lax.axis_index("s")
    r0 = s * ROWS_PER_TILE
    pltpu.sync_copy(zeros_hbm.at[pl.ds(r0, ROWS_PER_TILE)],
                    acc_sh.at[pl.ds(r0, ROWS_PER_TILE)])
    pltpu.sync_copy(ones_hbm, ones_v)
    plsc.subcore_barrier()

    kc = ones_v.shape[0]
    ept = E // NS
    base = s * ept
    nchunks = ept // kc

    def step(j, carry):
        off = c * E + base + j * kc
        pltpu.sync_copy(ei_hbm.at[pl.ds(off, kc)], idx_v)
        pltpu.sync_copy(ones_v, acc_sh.at[idx_v], add=True)
        return carry

    lax.fori_loop(0, nchunks, step, 0)
    plsc.subcore_barrier()
    pltpu.sync_copy(acc_sh.at[pl.ds(r0, ROWS_PER_TILE)],
                    out_hbm.at[c, pl.ds(r0, ROWS_PER_TILE)])


_deg_kernel = functools.partial(
    pl.kernel,
    out_type=jax.ShapeDtypeStruct((NC, NPAD, D), jnp.float32),
    mesh=plsc.VectorSubcoreMesh(**_MESH),
    scratch_types=[
        pltpu.VMEM_SHARED((NPAD, D), jnp.float32),
        pltpu.VMEM((KC,), jnp.int32),
        pltpu.VMEM((KC, D), jnp.float32),
        pltpu.SemaphoreType.DMA,
    ],
)(_deg_body)


def _mp_body(h_hbm, ei_hbm, zeros_hbm, out_hbm,
             acc_sh, src_v, dst_v, rows_v, sem):
    c = lax.axis_index("c")
    s = lax.axis_index("s")
    r0 = s * ROWS_PER_TILE
    pltpu.sync_copy(zeros_hbm.at[pl.ds(r0, ROWS_PER_TILE)],
                    acc_sh.at[pl.ds(r0, ROWS_PER_TILE)])
    plsc.subcore_barrier()

    ept = E // (NC * NS)
    base = (c * NS + s) * ept
    nchunks = ept // KC

    def step(j, carry):
        off = base + j * KC
        pltpu.sync_copy(ei_hbm.at[pl.ds(off, KC)], src_v)
        pltpu.sync_copy(ei_hbm.at[pl.ds(E + off, KC)], dst_v)
        pltpu.async_copy(h_hbm.at[src_v], rows_v, sem).wait()
        pltpu.sync_copy(rows_v, acc_sh.at[dst_v], add=True)
        return carry

    lax.fori_loop(0, nchunks, step, 0)
    plsc.subcore_barrier()
    pltpu.sync_copy(acc_sh.at[pl.ds(r0, ROWS_PER_TILE)],
                    out_hbm.at[c, pl.ds(r0, ROWS_PER_TILE)])


_mp = functools.partial(
    pl.kernel,
    out_type=jax.ShapeDtypeStruct((NC, NPAD, D), jnp.float32),
    mesh=plsc.VectorSubcoreMesh(**_MESH),
    scratch_types=[
        pltpu.VMEM_SHARED((NPAD, D), jnp.float32),
        pltpu.VMEM((KC,), jnp.int32),
        pltpu.VMEM((KC,), jnp.int32),
        pltpu.VMEM((KC, D), jnp.float32),
        pltpu.SemaphoreType.DMA,
    ],
)(_mp_body)


def _mm_body(x_ref, w_ref, o_ref):
    o_ref[...] = jnp.dot(x_ref[...], w_ref[...],
                         preferred_element_type=jnp.float32)


def _prep_body(xw_ref, degp_ref, o_h, o_ns, o_nd):
    degp = degp_ref[...]
    deg_out = degp[0, :N, 0]
    deg_in = degp[1, :N, 0]
    ns = jnp.where(deg_out > 0, lax.rsqrt(deg_out), 0.0)[:, None]
    nd = jnp.where(deg_in > 0, lax.rsqrt(deg_in), 0.0)[:, None]
    o_h[...] = xw_ref[...] * ns
    o_ns[...] = ns
    o_nd[...] = nd


def _mid_body(aggp_ref, nd_ref, b_ref, g_ref, be_ref, ns_ref, w_ref, o_ref):
    aggp = aggp_ref[...]
    agg = aggp[0, :N, :] + aggp[1, :N, :]
    t = agg * nd_ref[...] + b_ref[...]
    mu = jnp.mean(t, axis=0, keepdims=True)
    var = jnp.mean((t - mu) * (t - mu), axis=0, keepdims=True)
    t = (t - mu) * lax.rsqrt(var + 1e-5) * g_ref[...] + be_ref[...]
    t = jnp.maximum(t, 0.0)
    o_ref[...] = jnp.dot(t * ns_ref[...], w_ref[...],
                         preferred_element_type=jnp.float32)


def _fin_body(aggp_ref, nd_ref, b_ref, o_ref):
    aggp = aggp_ref[...]
    agg = aggp[0, :N, :NCLS] + aggp[1, :N, :NCLS]
    o_ref[...] = agg * nd_ref[...] + b_ref[...]


def _tc(body, out_shape, *args):
    return pl.pallas_call(body, out_shape=out_shape)(*args)


def kernel(x, edge_index, W0, b0, g0, be0, W1, b1, g1, be1, W2, b2):
    f32 = jnp.float32
    ei = edge_index.reshape(2 * E)
    ones_kc = jnp.ones((KC, D), f32)
    zeros_d = jnp.zeros((NPAD, D), f32)
    W2p = jnp.pad(W2, ((0, 0), (0, D - NCLS)))

    degp = _deg_kernel(ei, ones_kc, zeros_d)
    xw = _tc(_mm_body, jax.ShapeDtypeStruct((N, D), f32), x, W0)
    h0, ns, nd = _tc(
        _prep_body,
        (jax.ShapeDtypeStruct((N, D), f32),
         jax.ShapeDtypeStruct((N, 1), f32),
         jax.ShapeDtypeStruct((N, 1), f32)),
        xw, degp)

    aggp0 = _mp(h0, ei, zeros_d)
    h1 = _tc(_mid_body, jax.ShapeDtypeStruct((N, D), f32),
             aggp0, nd, b0.reshape(1, D), g0.reshape(1, D),
             be0.reshape(1, D), ns, W1)

    aggp1 = _mp(h1, ei, zeros_d)
    h2 = _tc(_mid_body, jax.ShapeDtypeStruct((N, D), f32),
             aggp1, nd, b1.reshape(1, D), g1.reshape(1, D),
             be1.reshape(1, D), ns, W2p)

    aggp2 = _mp(h2, ei, zeros_d)
    out = _tc(_fin_body, jax.ShapeDtypeStruct((N, NCLS), f32),
              aggp2, nd, b2.reshape(1, NCLS))
    return out


# speedup vs baseline: 8.3823x; 1.2759x over previous
import functools

import jax
import jax.numpy as jnp
from jax import lax
from jax.experimental import pallas as pl
from jax.experimental.pallas import tpu as pltpu
from jax.experimental.pallas import tpu_sc as plsc

N = 10000
E = 320000
D = 128
NCLS = 40
NPAD = 10240
NC, NS = 2, 16
KD = 2000
KCMP = 80
ROWS_PER_TILE = NPAD // NS

_MESH = dict(core_axis_name="c", subcore_axis_name="s")


HROWS = NPAD // D


def _deg_body(ei_hbm, zeros_hbm, out_hbm, hs, hd, src_v, dst_v,
              iota_s, iota_d, acc_sh):
    c = lax.axis_index("c")
    s = lax.axis_index("s")
    pltpu.sync_copy(zeros_hbm.at[pl.ds(0, HROWS)], hs)
    pltpu.sync_copy(zeros_hbm.at[pl.ds(0, HROWS)], hd)

    @pl.when(s < 2 * HROWS // 16)
    def _():
        pltpu.sync_copy(zeros_hbm.at[pl.ds(0, 16)],
                        acc_sh.at[pl.ds(s * 16, 16)])
    for k in range(HROWS // 16):
        base16 = lax.iota(jnp.int32, 16) + (16 * k)
        iota_s[pl.ds(16 * k, 16)] = base16
        iota_d[pl.ds(16 * k, 16)] = base16 + HROWS
    plsc.subcore_barrier()

    ept = E // (NC * NS)
    base = (c * NS + s) * ept

    def chunk(j, carry):
        off = base + j * KD
        pltpu.sync_copy(ei_hbm.at[pl.ds(off, KD)], src_v)
        pltpu.sync_copy(ei_hbm.at[pl.ds(E + off, KD)], dst_v)

        def inner(i, carry2):
            v = src_v[pl.ds(i * 16, 16)]
            cnt, last = plsc.scan_count(v)
            plsc.addupdate_scatter(
                hs, [v >> 7, v & 127], cnt.astype(jnp.float32), mask=last)
            w = dst_v[pl.ds(i * 16, 16)]
            cnt2, last2 = plsc.scan_count(w)
            plsc.addupdate_scatter(
                hd, [w >> 7, w & 127], cnt2.astype(jnp.float32), mask=last2)
            return carry2

        lax.fori_loop(0, KD // 16, inner, 0)
        return carry

    lax.fori_loop(0, ept // KD, chunk, 0)

    pltpu.sync_copy(hs, acc_sh.at[iota_s], add=True)
    pltpu.sync_copy(hd, acc_sh.at[iota_d], add=True)
    plsc.subcore_barrier()

    @pl.when(s < 2 * HROWS // 16)
    def _():
        pltpu.sync_copy(acc_sh.at[pl.ds(s * 16, 16)],
                        out_hbm.at[c, pl.ds(s * 16, 16)])


_deg_kernel = functools.partial(
    pl.kernel,
    out_type=jax.ShapeDtypeStruct((NC, 2 * HROWS, D), jnp.float32),
    mesh=plsc.VectorSubcoreMesh(**_MESH),
    scratch_types=[
        pltpu.VMEM((HROWS, D), jnp.float32),
        pltpu.VMEM((HROWS, D), jnp.float32),
        pltpu.VMEM((KD,), jnp.int32),
        pltpu.VMEM((KD,), jnp.int32),
        pltpu.VMEM((HROWS,), jnp.int32),
        pltpu.VMEM((HROWS,), jnp.int32),
        pltpu.VMEM_SHARED((2 * HROWS, D), jnp.float32),
    ],
    compiler_params=pltpu.CompilerParams(needs_layout_passes=False),
)(_deg_body)


def _mp_body(h_hbm, ei_hbm, zeros_hbm, out_hbm, acc_sh,
             src_a, src_b, dst_a, dst_b, rows_a, rows_b, sem_a, sem_b):
    c = lax.axis_index("c")
    s = lax.axis_index("s")
    r0 = s * ROWS_PER_TILE
    pltpu.sync_copy(zeros_hbm.at[pl.ds(r0, ROWS_PER_TILE)],
                    acc_sh.at[pl.ds(r0, ROWS_PER_TILE)])
    plsc.subcore_barrier()

    ept = E // (NC * NS)
    base = (c * NS + s) * ept
    nchunks = ept // KCMP

    def start_gather(off, src_v, rows_v, sem):
        pltpu.sync_copy(ei_hbm.at[pl.ds(off, KCMP)], src_v)
        pltpu.async_copy(h_hbm.at[src_v], rows_v, sem)

    def finish_scatter(off, src_v, dst_v, rows_v, sem):
        pltpu.make_async_copy(h_hbm.at[src_v], rows_v, sem).wait()
        pltpu.sync_copy(ei_hbm.at[pl.ds(E + off, KCMP)], dst_v)
        pltpu.sync_copy(rows_v, acc_sh.at[dst_v], add=True)

    start_gather(base, src_a, rows_a, sem_a)

    def step(i, carry):
        o0 = base + (2 * i) * KCMP
        start_gather(o0 + KCMP, src_b, rows_b, sem_b)
        finish_scatter(o0, src_a, dst_a, rows_a, sem_a)
        start_gather(o0 + 2 * KCMP, src_a, rows_a, sem_a)
        finish_scatter(o0 + KCMP, src_b, dst_b, rows_b, sem_b)
        return carry

    lax.fori_loop(0, (nchunks - 1) // 2, step, 0)
    finish_scatter(base + (nchunks - 1) * KCMP, src_a, dst_a, rows_a, sem_a)

    plsc.subcore_barrier()
    pltpu.sync_copy(acc_sh.at[pl.ds(r0, ROWS_PER_TILE)],
                    out_hbm.at[c, pl.ds(r0, ROWS_PER_TILE)])


_mp = functools.partial(
    pl.kernel,
    out_type=jax.ShapeDtypeStruct((NC, NPAD, D), jnp.float32),
    mesh=plsc.VectorSubcoreMesh(**_MESH),
    scratch_types=[
        pltpu.VMEM_SHARED((NPAD, D), jnp.float32),
        pltpu.VMEM((KCMP,), jnp.int32),
        pltpu.VMEM((KCMP,), jnp.int32),
        pltpu.VMEM((KCMP,), jnp.int32),
        pltpu.VMEM((KCMP,), jnp.int32),
        pltpu.VMEM((KCMP, D), jnp.float32),
        pltpu.VMEM((KCMP, D), jnp.float32),
        pltpu.SemaphoreType.DMA,
        pltpu.SemaphoreType.DMA,
    ],
)(_mp_body)


def _mm_body(x_ref, w_ref, o_ref):
    o_ref[...] = jnp.dot(x_ref[...], w_ref[...],
                         preferred_element_type=jnp.float32)


def _prep_body(xw_ref, degp_ref, o_h, o_ns, o_nd):
    degp_full = degp_ref[...]
    degp = degp_full[0] + degp_full[1]
    deg_out = degp[:HROWS].reshape(NPAD)[:N]
    deg_in = degp[HROWS:].reshape(NPAD)[:N]
    ns = jnp.where(deg_out > 0, lax.rsqrt(deg_out), 0.0)[:, None]
    nd = jnp.where(deg_in > 0, lax.rsqrt(deg_in), 0.0)[:, None]
    o_h[...] = xw_ref[...] * ns
    o_ns[...] = ns
    o_nd[...] = nd


def _mid_body(aggp_ref, nd_ref, b_ref, g_ref, be_ref, ns_ref, w_ref, o_ref):
    aggp = aggp_ref[...]
    agg = aggp[0, :N, :] + aggp[1, :N, :]
    t = agg * nd_ref[...] + b_ref[...]
    mu = jnp.mean(t, axis=0, keepdims=True)
    var = jnp.mean((t - mu) * (t - mu), axis=0, keepdims=True)
    t = (t - mu) * lax.rsqrt(var + 1e-5) * g_ref[...] + be_ref[...]
    t = jnp.maximum(t, 0.0)
    o_ref[...] = jnp.dot(t * ns_ref[...], w_ref[...],
                         preferred_element_type=jnp.float32)


def _fin_body(aggp_ref, nd_ref, b_ref, o_ref):
    aggp = aggp_ref[...]
    agg = aggp[0, :N, :NCLS] + aggp[1, :N, :NCLS]
    o_ref[...] = agg * nd_ref[...] + b_ref[...]


def _tc(body, out_shape, *args):
    return pl.pallas_call(body, out_shape=out_shape)(*args)


def kernel(x, edge_index, W0, b0, g0, be0, W1, b1, g1, be1, W2, b2):
    f32 = jnp.float32
    ei = edge_index.reshape(2 * E)
    zeros_d = jnp.zeros((NPAD, D), f32)
    W2p = jnp.pad(W2, ((0, 0), (0, D - NCLS)))

    degp = _deg_kernel(ei, zeros_d)
    xw = _tc(_mm_body, jax.ShapeDtypeStruct((N, D), f32), x, W0)
    h0, ns, nd = _tc(
        _prep_body,
        (jax.ShapeDtypeStruct((N, D), f32),
         jax.ShapeDtypeStruct((N, 1), f32),
         jax.ShapeDtypeStruct((N, 1), f32)),
        xw, degp)

    aggp0 = _mp(h0, ei, zeros_d)
    h1 = _tc(_mid_body, jax.ShapeDtypeStruct((N, D), f32),
             aggp0, nd, b0.reshape(1, D), g0.reshape(1, D),
             be0.reshape(1, D), ns, W1)

    aggp1 = _mp(h1, ei, zeros_d)
    h2 = _tc(_mid_body, jax.ShapeDtypeStruct((N, D), f32),
             aggp1, nd, b1.reshape(1, D), g1.reshape(1, D),
             be1.reshape(1, D), ns, W2p)

    aggp2 = _mp(h2, ei, zeros_d)
    out = _tc(_fin_body, jax.ShapeDtypeStruct((N, NCLS), f32),
              aggp2, nd, b2.reshape(1, NCLS))
    return out
